# baseline (speedup 1.0000x reference)
import functools

import jax
import jax.numpy as jnp
from jax import lax
from jax.experimental import pallas as pl
from jax.experimental.pallas import tpu as pltpu
from jax.experimental.pallas import tpu_sc as plsc

B = 1024
EMB = 32
L_IDS = 50
L_CIDS = 20
N_IDS = 100000

NW = 32
CHUNK = 80

BT = 64
TN = 2048
NT = 49
NP = NT * TN

_HIGH = lax.Precision.HIGHEST


def _sc_gather(ids_flat, cids_flat, table_dae, table_cnn):
    n_dae = ids_flat.shape[0]
    n_cnn = cids_flat.shape[0]
    per_dae = n_dae // NW
    per_cnn = n_cnn // NW
    mesh = plsc.VectorSubcoreMesh(core_axis_name="c", subcore_axis_name="s")

    @functools.partial(
        pl.kernel,
        out_type=(
            jax.ShapeDtypeStruct((n_dae, EMB), jnp.float32),
            jax.ShapeDtypeStruct((n_cnn, EMB), jnp.float32),
        ),
        mesh=mesh,
        scratch_types=[
            pltpu.VMEM((per_dae,), jnp.int32),
            pltpu.VMEM((per_dae, EMB), jnp.float32),
            pltpu.SemaphoreType.DMA,
        ],
        compiler_params=pltpu.CompilerParams(use_tc_tiling_on_sc=False),
    )
    def gather_kernel(ids_hbm, cids_hbm, tdae_hbm, tcnn_hbm,
                      odae_hbm, ocnn_hbm, idx_v, rows_v, sem):
        wid = lax.axis_index("s") * 2 + lax.axis_index("c")

        base = wid * per_dae
        pltpu.sync_copy(ids_hbm.at[pl.ds(base, per_dae)], idx_v)

        @pl.loop(0, per_dae // CHUNK)
        def _(c):
            pltpu.async_copy(
                tdae_hbm.at[idx_v.at[pl.ds(c * CHUNK, CHUNK)]],
                rows_v.at[pl.ds(c * CHUNK, CHUNK)], sem)

        @pl.loop(0, per_dae // CHUNK)
        def _(c):
            pltpu.make_async_copy(
                tdae_hbm.at[idx_v.at[pl.ds(c * CHUNK, CHUNK)]],
                rows_v.at[pl.ds(c * CHUNK, CHUNK)], sem).wait()

        pltpu.sync_copy(rows_v, odae_hbm.at[pl.ds(base, per_dae)])

        base2 = wid * per_cnn
        pltpu.sync_copy(cids_hbm.at[pl.ds(base2, per_cnn)],
                        idx_v.at[pl.ds(0, per_cnn)])

        @pl.loop(0, per_cnn // CHUNK)
        def _(c):
            pltpu.async_copy(
                tcnn_hbm.at[idx_v.at[pl.ds(c * CHUNK, CHUNK)]],
                rows_v.at[pl.ds(c * CHUNK, CHUNK)], sem)

        @pl.loop(0, per_cnn // CHUNK)
        def _(c):
            pltpu.make_async_copy(
                tcnn_hbm.at[idx_v.at[pl.ds(c * CHUNK, CHUNK)]],
                rows_v.at[pl.ds(c * CHUNK, CHUNK)], sem).wait()

        pltpu.sync_copy(rows_v.at[pl.ds(0, per_cnn)],
                        ocnn_hbm.at[pl.ds(base2, per_cnn)])

    return gather_kernel(ids_flat, cids_flat, table_dae, table_cnn)


def _decode_body(we_ref, wf_ref, m_ref):
    m128 = lax.dot_general(we_ref[...], wf_ref[...],
                           (((0,), (0,)), ((), ())),
                           preferred_element_type=jnp.float32,
                           precision=_HIGH)
    m_ref[...] = (m128[0:32, 0:32] + m128[32:64, 32:64]
                  + m128[64:96, 64:96] + m128[96:128, 96:128])


def _decode(W_emb_dae, W_dae_ff1):
    return pl.pallas_call(
        _decode_body,
        out_shape=jax.ShapeDtypeStruct((EMB, EMB), jnp.float32),
    )(W_emb_dae.reshape(N_IDS // 4, 4 * EMB),
      W_dae_ff1.reshape(N_IDS // 4, 4 * EMB))


def _seg_sum(flat, length):
    sel = (lax.broadcasted_iota(jnp.int32, (length * EMB, EMB), 0) % EMB
           == lax.broadcasted_iota(jnp.int32, (length * EMB, EMB), 1)
           ).astype(jnp.float32)
    return jnp.dot(flat, sel, preferred_element_type=jnp.float32,
                   precision=_HIGH)


def _prep_body(gd_ref, gc_ref, m_ref, bd_ref, wc_ref, bc_ref,
               yd_ref, yc_ref):
    sd = _seg_sum(gd_ref[...], L_IDS)
    x = jnp.maximum(sd, 0.0)
    yd = jnp.dot(x, m_ref[...], preferred_element_type=jnp.float32,
                 precision=_HIGH) + bd_ref[...]
    yd_ref[...] = jnp.maximum(yd, 0.0)

    sc = _seg_sum(gc_ref[...], L_CIDS)
    c2 = jnp.dot(sc, wc_ref[...], preferred_element_type=jnp.float32,
                 precision=_HIGH) + bc_ref[...]
    c2 = jnp.maximum(c2, 0.0)
    cmax = jnp.max(c2, axis=1, keepdims=True)
    ce = jnp.exp(c2 - cmax)
    yc_ref[...] = ce / jnp.sum(ce, axis=1, keepdims=True)


def _prep(g_dae, g_cnn, m32, b_dae, W_cnn_ff1, b_cnn):
    return pl.pallas_call(
        _prep_body,
        out_shape=(
            jax.ShapeDtypeStruct((B, EMB), jnp.float32),
            jax.ShapeDtypeStruct((B, EMB), jnp.float32),
        ),
    )(g_dae, g_cnn, m32, b_dae, W_cnn_ff1, b_cnn)


def _head_body(h_ref, w_hbm, b_hbm, o_ref, w_ref, b_ref, slab, mref, sref,
               sem):
    i = pl.program_id(0)
    p = pl.program_id(1)
    j = pl.program_id(2)
    col0 = j * TN

    @pl.when((i == 0) & (p == 0) & (j == 0))
    def _():
        cw = pltpu.make_async_copy(w_hbm, w_ref, sem)
        cw.start()
        cw.wait()
        cb = pltpu.make_async_copy(b_hbm, b_ref, sem)
        cb.start()
        cb.wait()

    @pl.when(p == 0)
    def _():
        z = jnp.dot(h_ref[...], w_ref[:, pl.ds(col0, TN)],
                    preferred_element_type=jnp.float32, precision=_HIGH)
        z = z + b_ref[0:1, pl.ds(col0, TN)]
        z = jnp.maximum(z, 0.0)
        valid = (col0 + lax.broadcasted_iota(jnp.int32, (BT, TN), 1)) < N_IDS
        z = jnp.where(valid, z, -3.0e38)
        slab[:, pl.ds(col0, TN)] = z
        tmax = jnp.max(z, axis=1, keepdims=True)
        prev = jnp.where(j == 0, -3.0e38, mref[:, 0:1])
        mref[:, 0:1] = jnp.maximum(prev, tmax)

    @pl.when(p == 1)
    def _():
        e = jnp.exp(slab[:, pl.ds(col0, TN)] - mref[:, 0:1])
        slab[:, pl.ds(col0, TN)] = e
        ts = jnp.sum(e, axis=1, keepdims=True)
        prev = jnp.where(j == 0, 0.0, sref[:, 0:1])
        sref[:, 0:1] = prev + ts

    @pl.when(p == 2)
    def _():
        o_ref[...] = slab[:, pl.ds(col0, TN)] * (1.0 / sref[:, 0:1])


def _head(h, W_ffp, b_ffp):
    grid = (B // BT, 3, NT)
    return pl.pallas_call(
        _head_body,
        grid=grid,
        in_specs=[
            pl.BlockSpec((BT, 64), lambda i, p, j: (i, 0)),
            pl.BlockSpec(memory_space=pl.ANY),
            pl.BlockSpec(memory_space=pl.ANY),
        ],
        out_specs=pl.BlockSpec((BT, TN), lambda i, p, j: (i, j * (p // 2))),
        out_shape=jax.ShapeDtypeStruct((B, N_IDS), jnp.float32),
        scratch_shapes=[
            pltpu.VMEM((64, NP), jnp.float32),
            pltpu.VMEM((1, NP), jnp.float32),
            pltpu.VMEM((BT, NP), jnp.float32),
            pltpu.VMEM((BT, 128), jnp.float32),
            pltpu.VMEM((BT, 128), jnp.float32),
            pltpu.SemaphoreType.DMA,
        ],
        compiler_params=pltpu.CompilerParams(
            dimension_semantics=("arbitrary", "arbitrary", "arbitrary"),
        ),
    )(h, W_ffp, b_ffp)


def kernel(ids, cids, W_emb_dae, W_dae_ff1, b_dae_ff1, W_emb_cnn,
           W_cnn_ff1, b_cnn_ff1, W_ff, b_ff):
    ids_flat = ids.reshape(-1).astype(jnp.int32)
    cids_flat = cids.reshape(-1).astype(jnp.int32)

    g_dae, g_cnn = _sc_gather(ids_flat, cids_flat, W_emb_dae, W_emb_cnn)
    m32 = _decode(W_emb_dae, W_dae_ff1)

    y_dae, y_cnn = _prep(
        g_dae.reshape(B, L_IDS * EMB),
        g_cnn.reshape(B, L_CIDS * EMB),
        m32,
        b_dae_ff1.reshape(1, EMB),
        W_cnn_ff1,
        b_cnn_ff1.reshape(1, EMB),
    )
    h = jnp.concatenate([y_dae, y_cnn], axis=1)

    W_ffp = jnp.pad(W_ff, ((0, 0), (0, NP - N_IDS)))
    b_ffp = jnp.pad(b_ff, (0, NP - N_IDS)).reshape(1, NP)
    return _head(h, W_ffp, b_ffp)


# speedup vs baseline: 1.2196x; 1.2196x over previous
import functools

import jax
import jax.numpy as jnp
from jax import lax
from jax.experimental import pallas as pl
from jax.experimental.pallas import tpu as pltpu
from jax.experimental.pallas import tpu_sc as plsc

B = 1024
EMB = 32
L_IDS = 50
L_CIDS = 20
N_IDS = 100000

NW = 32
CHUNK = 80

BT = 64
TN = 2048
NT = 49
NP = NT * TN

_HIGH = lax.Precision.HIGHEST


def _sc_gather(ids_flat, cids_flat, table_dae, table_cnn):
    n_dae = ids_flat.shape[0]
    n_cnn = cids_flat.shape[0]
    per_dae = n_dae // NW
    per_cnn = n_cnn // NW
    mesh = plsc.VectorSubcoreMesh(core_axis_name="c", subcore_axis_name="s")

    @functools.partial(
        pl.kernel,
        out_type=(
            jax.ShapeDtypeStruct((n_dae, EMB), jnp.float32),
            jax.ShapeDtypeStruct((n_cnn, EMB), jnp.float32),
        ),
        mesh=mesh,
        scratch_types=[
            pltpu.VMEM((per_dae,), jnp.int32),
            pltpu.VMEM((per_dae, EMB), jnp.float32),
            pltpu.SemaphoreType.DMA,
        ],
        compiler_params=pltpu.CompilerParams(use_tc_tiling_on_sc=False),
    )
    def gather_kernel(ids_hbm, cids_hbm, tdae_hbm, tcnn_hbm,
                      odae_hbm, ocnn_hbm, idx_v, rows_v, sem):
        wid = lax.axis_index("s") * 2 + lax.axis_index("c")

        base = wid * per_dae
        pltpu.sync_copy(ids_hbm.at[pl.ds(base, per_dae)], idx_v)

        @pl.loop(0, per_dae // CHUNK)
        def _(c):
            pltpu.async_copy(
                tdae_hbm.at[idx_v.at[pl.ds(c * CHUNK, CHUNK)]],
                rows_v.at[pl.ds(c * CHUNK, CHUNK)], sem)

        @pl.loop(0, per_dae // CHUNK)
        def _(c):
            pltpu.make_async_copy(
                tdae_hbm.at[idx_v.at[pl.ds(c * CHUNK, CHUNK)]],
                rows_v.at[pl.ds(c * CHUNK, CHUNK)], sem).wait()

        pltpu.sync_copy(rows_v, odae_hbm.at[pl.ds(base, per_dae)])

        base2 = wid * per_cnn
        pltpu.sync_copy(cids_hbm.at[pl.ds(base2, per_cnn)],
                        idx_v.at[pl.ds(0, per_cnn)])

        @pl.loop(0, per_cnn // CHUNK)
        def _(c):
            pltpu.async_copy(
                tcnn_hbm.at[idx_v.at[pl.ds(c * CHUNK, CHUNK)]],
                rows_v.at[pl.ds(c * CHUNK, CHUNK)], sem)

        @pl.loop(0, per_cnn // CHUNK)
        def _(c):
            pltpu.make_async_copy(
                tcnn_hbm.at[idx_v.at[pl.ds(c * CHUNK, CHUNK)]],
                rows_v.at[pl.ds(c * CHUNK, CHUNK)], sem).wait()

        pltpu.sync_copy(rows_v.at[pl.ds(0, per_cnn)],
                        ocnn_hbm.at[pl.ds(base2, per_cnn)])

    return gather_kernel(ids_flat, cids_flat, table_dae, table_cnn)


def _decode_body(we_ref, wf_ref, m_ref):
    m128 = lax.dot_general(we_ref[...], wf_ref[...],
                           (((0,), (0,)), ((), ())),
                           preferred_element_type=jnp.float32,
                           precision=_HIGH)
    m_ref[...] = (m128[0:32, 0:32] + m128[32:64, 32:64]
                  + m128[64:96, 64:96] + m128[96:128, 96:128])


def _decode(W_emb_dae, W_dae_ff1):
    return pl.pallas_call(
        _decode_body,
        out_shape=jax.ShapeDtypeStruct((EMB, EMB), jnp.float32),
    )(W_emb_dae.reshape(N_IDS // 4, 4 * EMB),
      W_dae_ff1.reshape(N_IDS // 4, 4 * EMB))


def _seg_sum(flat, length):
    sel = (lax.broadcasted_iota(jnp.int32, (length * EMB, EMB), 0) % EMB
           == lax.broadcasted_iota(jnp.int32, (length * EMB, EMB), 1)
           ).astype(jnp.float32)
    return jnp.dot(flat, sel, preferred_element_type=jnp.float32,
                   precision=_HIGH)


def _prep_body(gd_ref, gc_ref, m_ref, bd_ref, wc_ref, bc_ref,
               yd_ref, yc_ref):
    sd = _seg_sum(gd_ref[...], L_IDS)
    x = jnp.maximum(sd, 0.0)
    yd = jnp.dot(x, m_ref[...], preferred_element_type=jnp.float32,
                 precision=_HIGH) + bd_ref[...]
    yd_ref[...] = jnp.maximum(yd, 0.0)

    sc = _seg_sum(gc_ref[...], L_CIDS)
    c2 = jnp.dot(sc, wc_ref[...], preferred_element_type=jnp.float32,
                 precision=_HIGH) + bc_ref[...]
    c2 = jnp.maximum(c2, 0.0)
    cmax = jnp.max(c2, axis=1, keepdims=True)
    ce = jnp.exp(c2 - cmax)
    yc_ref[...] = ce / jnp.sum(ce, axis=1, keepdims=True)


def _prep(g_dae, g_cnn, m32, b_dae, W_cnn_ff1, b_cnn):
    return pl.pallas_call(
        _prep_body,
        out_shape=(
            jax.ShapeDtypeStruct((B, EMB), jnp.float32),
            jax.ShapeDtypeStruct((B, EMB), jnp.float32),
        ),
    )(g_dae, g_cnn, m32, b_dae, W_cnn_ff1, b_cnn)


def _head_body(h_ref, w_ref, b_ref, o_ref, slab, mref, sref):
    p = pl.program_id(1)
    j = pl.program_id(2)
    col0 = j * TN

    @pl.when(p == 0)
    def _():
        z = jnp.dot(h_ref[...], w_ref[:, pl.ds(col0, TN)],
                    preferred_element_type=jnp.float32)
        z = z + b_ref[0:1, pl.ds(col0, TN)]
        z = jnp.maximum(z, 0.0)
        valid = (col0 + lax.broadcasted_iota(jnp.int32, (BT, TN), 1)) < N_IDS
        z = jnp.where(valid, z, -3.0e38)
        slab[:, pl.ds(col0, TN)] = z
        tmax = jnp.max(z, axis=1, keepdims=True)
        prev = jnp.where(j == 0, -3.0e38, mref[:, 0:1])
        mref[:, 0:1] = jnp.maximum(prev, tmax)

    @pl.when(p == 1)
    def _():
        e = jnp.exp(slab[:, pl.ds(col0, TN)] - mref[:, 0:1])
        slab[:, pl.ds(col0, TN)] = e
        ts = jnp.sum(e, axis=1, keepdims=True)
        prev = jnp.where(j == 0, 0.0, sref[:, 0:1])
        sref[:, 0:1] = prev + ts

    @pl.when(p == 2)
    def _():
        o_ref[...] = slab[:, pl.ds(col0, TN)] * (1.0 / sref[:, 0:1])


def _head(h, W_ffp, b_ffp):
    grid = (B // BT, 3, NT)
    return pl.pallas_call(
        _head_body,
        grid=grid,
        in_specs=[
            pl.BlockSpec((BT, 64), lambda i, p, j: (i, 0)),
            pl.BlockSpec((64, NP), lambda i, p, j: (0, 0)),
            pl.BlockSpec((1, NP), lambda i, p, j: (0, 0)),
        ],
        out_specs=pl.BlockSpec((BT, TN), lambda i, p, j: (i, j * (p // 2))),
        out_shape=jax.ShapeDtypeStruct((B, N_IDS), jnp.float32),
        scratch_shapes=[
            pltpu.VMEM((BT, NP), jnp.float32),
            pltpu.VMEM((BT, 128), jnp.float32),
            pltpu.VMEM((BT, 128), jnp.float32),
        ],
        compiler_params=pltpu.CompilerParams(
            dimension_semantics=("parallel", "arbitrary", "arbitrary"),
        ),
    )(h, W_ffp, b_ffp)


def kernel(ids, cids, W_emb_dae, W_dae_ff1, b_dae_ff1, W_emb_cnn,
           W_cnn_ff1, b_cnn_ff1, W_ff, b_ff):
    ids_flat = ids.reshape(-1).astype(jnp.int32)
    cids_flat = cids.reshape(-1).astype(jnp.int32)

    g_dae, g_cnn = _sc_gather(ids_flat, cids_flat, W_emb_dae, W_emb_cnn)
    m32 = _decode(W_emb_dae, W_dae_ff1)

    y_dae, y_cnn = _prep(
        g_dae.reshape(B, L_IDS * EMB),
        g_cnn.reshape(B, L_CIDS * EMB),
        m32,
        b_dae_ff1.reshape(1, EMB),
        W_cnn_ff1,
        b_cnn_ff1.reshape(1, EMB),
    )
    h = jnp.concatenate([y_dae, y_cnn], axis=1).astype(jnp.bfloat16)

    W_ffp = jnp.pad(W_ff, ((0, 0), (0, NP - N_IDS))).astype(jnp.bfloat16)
    b_ffp = jnp.pad(b_ff, (0, NP - N_IDS)).reshape(1, NP)
    return _head(h, W_ffp, b_ffp)


# speedup vs baseline: 1.9204x; 1.5746x over previous
import functools

import jax
import jax.numpy as jnp
from jax import lax
from jax.experimental import pallas as pl
from jax.experimental.pallas import tpu as pltpu
from jax.experimental.pallas import tpu_sc as plsc

B = 1024
EMB = 32
L_IDS = 50
L_CIDS = 20
N_IDS = 100000

NW = 32
CHUNK = 80

BT = 512
TN = 4096
NT = 25
NP = NT * TN

_HIGH = lax.Precision.HIGHEST


def _sc_gather(ids_flat, cids_flat, table_dae, table_cnn):
    n_dae = ids_flat.shape[0]
    n_cnn = cids_flat.shape[0]
    per_dae = n_dae // NW
    per_cnn = n_cnn // NW
    mesh = plsc.VectorSubcoreMesh(core_axis_name="c", subcore_axis_name="s")

    @functools.partial(
        pl.kernel,
        out_type=(
            jax.ShapeDtypeStruct((n_dae, EMB), jnp.float32),
            jax.ShapeDtypeStruct((n_cnn, EMB), jnp.float32),
        ),
        mesh=mesh,
        scratch_types=[
            pltpu.VMEM((per_dae,), jnp.int32),
            pltpu.VMEM((per_dae, EMB), jnp.float32),
            pltpu.SemaphoreType.DMA,
        ],
        compiler_params=pltpu.CompilerParams(use_tc_tiling_on_sc=False),
    )
    def gather_kernel(ids_hbm, cids_hbm, tdae_hbm, tcnn_hbm,
                      odae_hbm, ocnn_hbm, idx_v, rows_v, sem):
        wid = lax.axis_index("s") * 2 + lax.axis_index("c")

        base = wid * per_dae
        pltpu.sync_copy(ids_hbm.at[pl.ds(base, per_dae)], idx_v)

        @pl.loop(0, per_dae // CHUNK)
        def _(c):
            pltpu.async_copy(
                tdae_hbm.at[idx_v.at[pl.ds(c * CHUNK, CHUNK)]],
                rows_v.at[pl.ds(c * CHUNK, CHUNK)], sem)

        @pl.loop(0, per_dae // CHUNK)
        def _(c):
            pltpu.make_async_copy(
                tdae_hbm.at[idx_v.at[pl.ds(c * CHUNK, CHUNK)]],
                rows_v.at[pl.ds(c * CHUNK, CHUNK)], sem).wait()

        pltpu.sync_copy(rows_v, odae_hbm.at[pl.ds(base, per_dae)])

        base2 = wid * per_cnn
        pltpu.sync_copy(cids_hbm.at[pl.ds(base2, per_cnn)],
                        idx_v.at[pl.ds(0, per_cnn)])

        @pl.loop(0, per_cnn // CHUNK)
        def _(c):
            pltpu.async_copy(
                tcnn_hbm.at[idx_v.at[pl.ds(c * CHUNK, CHUNK)]],
                rows_v.at[pl.ds(c * CHUNK, CHUNK)], sem)

        @pl.loop(0, per_cnn // CHUNK)
        def _(c):
            pltpu.make_async_copy(
                tcnn_hbm.at[idx_v.at[pl.ds(c * CHUNK, CHUNK)]],
                rows_v.at[pl.ds(c * CHUNK, CHUNK)], sem).wait()

        pltpu.sync_copy(rows_v.at[pl.ds(0, per_cnn)],
                        ocnn_hbm.at[pl.ds(base2, per_cnn)])

    return gather_kernel(ids_flat, cids_flat, table_dae, table_cnn)


def _decode_body(we_ref, wf_ref, m_ref):
    m128 = lax.dot_general(we_ref[...], wf_ref[...],
                           (((0,), (0,)), ((), ())),
                           preferred_element_type=jnp.float32,
                           precision=_HIGH)
    m_ref[...] = (m128[0:32, 0:32] + m128[32:64, 32:64]
                  + m128[64:96, 64:96] + m128[96:128, 96:128])


def _decode(W_emb_dae, W_dae_ff1):
    return pl.pallas_call(
        _decode_body,
        out_shape=jax.ShapeDtypeStruct((EMB, EMB), jnp.float32),
    )(W_emb_dae.reshape(N_IDS // 4, 4 * EMB),
      W_dae_ff1.reshape(N_IDS // 4, 4 * EMB))


def _seg_sum(flat, length):
    sel = (lax.broadcasted_iota(jnp.int32, (length * EMB, EMB), 0) % EMB
           == lax.broadcasted_iota(jnp.int32, (length * EMB, EMB), 1)
           ).astype(jnp.float32)
    return jnp.dot(flat, sel, preferred_element_type=jnp.float32,
                   precision=_HIGH)


def _prep_body(gd_ref, gc_ref, m_ref, bd_ref, wc_ref, bc_ref,
               yd_ref, yc_ref):
    sd = _seg_sum(gd_ref[...], L_IDS)
    x = jnp.maximum(sd, 0.0)
    yd = jnp.dot(x, m_ref[...], preferred_element_type=jnp.float32,
                 precision=_HIGH) + bd_ref[...]
    yd_ref[...] = jnp.maximum(yd, 0.0)

    sc = _seg_sum(gc_ref[...], L_CIDS)
    c2 = jnp.dot(sc, wc_ref[...], preferred_element_type=jnp.float32,
                 precision=_HIGH) + bc_ref[...]
    c2 = jnp.maximum(c2, 0.0)
    cmax = jnp.max(c2, axis=1, keepdims=True)
    ce = jnp.exp(c2 - cmax)
    yc_ref[...] = ce / jnp.sum(ce, axis=1, keepdims=True)


def _prep(g_dae, g_cnn, m32, b_dae, W_cnn_ff1, b_cnn):
    return pl.pallas_call(
        _prep_body,
        out_shape=(
            jax.ShapeDtypeStruct((B, EMB), jnp.float32),
            jax.ShapeDtypeStruct((B, EMB), jnp.float32),
        ),
    )(g_dae, g_cnn, m32, b_dae, W_cnn_ff1, b_cnn)


def _head_body(h_ref, w_ref, b_ref, bp_ref, o_ref, mref, sref):
    p = pl.program_id(1)
    j = pl.program_id(2)

    z = jnp.dot(h_ref[...], w_ref[...], preferred_element_type=jnp.float32)
    z = jnp.maximum(z + b_ref[...], 0.0)

    @pl.when(p == 0)
    def _():
        zm = z + bp_ref[...]
        tmax = jnp.max(zm, axis=1, keepdims=True)
        m_old = jnp.where(j == 0, -3.0e38, mref[:, 0:1])
        m_new = jnp.maximum(m_old, tmax)
        e = jnp.exp(zm - m_new)
        ts = jnp.sum(e, axis=1, keepdims=True)
        s_old = jnp.where(j == 0, 0.0, sref[:, 0:1])
        sref[:, 0:1] = s_old * jnp.exp(m_old - m_new) + ts
        mref[:, 0:1] = m_new

    @pl.when(p == 1)
    def _():
        o_ref[...] = jnp.exp(z - mref[:, 0:1]) * (1.0 / sref[:, 0:1])


def _head(h, W_ffp, b_ffp, bpost):
    grid = (B // BT, 2, NT)
    return pl.pallas_call(
        _head_body,
        grid=grid,
        in_specs=[
            pl.BlockSpec((BT, 64), lambda i, p, j: (i, 0)),
            pl.BlockSpec((64, TN), lambda i, p, j: (0, j)),
            pl.BlockSpec((1, TN), lambda i, p, j: (0, j)),
            pl.BlockSpec((1, TN), lambda i, p, j: (0, j)),
        ],
        out_specs=pl.BlockSpec((BT, TN), lambda i, p, j: (i, j * p)),
        out_shape=jax.ShapeDtypeStruct((B, N_IDS), jnp.float32),
        scratch_shapes=[
            pltpu.VMEM((BT, 128), jnp.float32),
            pltpu.VMEM((BT, 128), jnp.float32),
        ],
        compiler_params=pltpu.CompilerParams(
            dimension_semantics=("parallel", "arbitrary", "arbitrary"),
        ),
    )(h, W_ffp, b_ffp, bpost)


def kernel(ids, cids, W_emb_dae, W_dae_ff1, b_dae_ff1, W_emb_cnn,
           W_cnn_ff1, b_cnn_ff1, W_ff, b_ff):
    ids_flat = ids.reshape(-1).astype(jnp.int32)
    cids_flat = cids.reshape(-1).astype(jnp.int32)

    g_dae, g_cnn = _sc_gather(ids_flat, cids_flat, W_emb_dae, W_emb_cnn)
    m32 = _decode(W_emb_dae, W_dae_ff1)

    y_dae, y_cnn = _prep(
        g_dae.reshape(B, L_IDS * EMB),
        g_cnn.reshape(B, L_CIDS * EMB),
        m32,
        b_dae_ff1.reshape(1, EMB),
        W_cnn_ff1,
        b_cnn_ff1.reshape(1, EMB),
    )
    h = jnp.concatenate([y_dae, y_cnn], axis=1).astype(jnp.bfloat16)

    W_ffp = jnp.pad(W_ff, ((0, 0), (0, NP - N_IDS))).astype(jnp.bfloat16)
    b_ffp = jnp.pad(b_ff, (0, NP - N_IDS)).reshape(1, NP)
    bpost = jnp.where(jnp.arange(NP) < N_IDS, 0.0, -3.0e38
                      ).astype(jnp.float32).reshape(1, NP)
    return _head(h, W_ffp, b_ffp, bpost)
